# baseline (speedup 1.0000x reference)
import functools

import jax
import jax.numpy as jnp
from jax import lax
from jax.experimental import pallas as pl
from jax.experimental.pallas import tpu as pltpu
from jax.experimental.pallas import tpu_sc as plsc

B, N, H, V = 16, 512, 16, 245
NP = N + 1
NC, NS, L = 2, 16, 16
NW = NC * NS
ROWS_W = (B * N) // NW
RCH = 8
NCH = ROWS_W // RCH
GPR = N // L
CW = RCH * NP
ST_H = 4632
SZ_N = CW + 8
SZ_L = CW + NP - 1
TW = V * H
HS = H + 1
IPREV, IMAIN, INEXT = 0, N, N + RCH * N
IBUF = N + RCH * N + N
SPTOT = B * N * N


def _body(sp_hbm, tab_hbm, out_hbm, tab_v, idx_v, stg_v, zbuf_v, sem_o,
          sem_i):
    cid = lax.axis_index("c")
    sid = lax.axis_index("s")
    wid = sid * NC + cid
    b = wid // 2
    half = wid % 2
    p0 = b * H

    iota = lax.iota(jnp.int32, L)
    zeros = jnp.zeros((L,), jnp.float32)
    izeros = jnp.zeros((L,), jnp.int32)

    pltpu.sync_copy(tab_hbm, stg_v.at[pl.ds(0, TW)])

    def repack(v, carry):
        row = stg_v[pl.ds(v * H, L)]
        plsc.store_scatter(tab_v, [v * HS + iota], row)
        return carry
    lax.fori_loop(0, V, repack, 0)

    for j in range(N // L):
        zbuf_v[pl.ds(j * L, L)] = zeros
        idx_v[pl.ds(IPREV + j * L, L)] = izeros
        idx_v[pl.ds(IBUF + IPREV + j * L, L)] = izeros

    @pl.when(wid == 0)
    def _():
        pltpu.sync_copy(zbuf_v, out_hbm.at[pl.ds(0, N)])

    row_base = half * ROWS_W
    idx_base = b * (N * N) + row_base * N

    def issue_idx(k, ibase):
        om = pl.multiple_of(idx_base + k * (RCH * N), 8)
        op = pl.multiple_of(jnp.maximum(om - N, 0), 8)
        on = pl.multiple_of(jnp.minimum(om + RCH * N, SPTOT - N), 8)
        pltpu.async_copy(
            sp_hbm.at[pl.ds(om, RCH * N)],
            idx_v.at[pl.ds(ibase + IMAIN, RCH * N)],
            sem_i,
        )
        pltpu.async_copy(
            sp_hbm.at[pl.ds(op, N)], idx_v.at[pl.ds(ibase + IPREV, N)], sem_i
        )
        pltpu.async_copy(
            sp_hbm.at[pl.ds(on, N)], idx_v.at[pl.ds(ibase + INEXT, N)], sem_i
        )

    def drain_idx():
        pltpu.make_async_copy(
            sp_hbm.at[pl.ds(0, RCH * N)], idx_v.at[pl.ds(IMAIN, RCH * N)],
            sem_i,
        ).wait()
        pltpu.make_async_copy(
            sp_hbm.at[pl.ds(0, N)], idx_v.at[pl.ds(IPREV, N)], sem_i
        ).wait()
        pltpu.make_async_copy(
            sp_hbm.at[pl.ds(0, N)], idx_v.at[pl.ds(INEXT, N)], sem_i
        ).wait()

    def drain_out():
        for _h in range(H):
            pltpu.make_async_copy(
                out_hbm.at[pl.ds(0, SZ_N)], stg_v.at[pl.ds(0, SZ_N)], sem_o
            ).wait()

    issue_idx(0, 0)

    def chunk(k, carry):
        r0 = row_base + k * RCH
        is_last = jnp.logical_and(half == 1, k == NCH - 1)
        ibase = (k % 2) * IBUF

        @pl.when(k > 0)
        def _():
            drain_out()

        drain_idx()

        @pl.when(k < NCH - 1)
        def _():
            issue_idx(k + 1, (1 - k % 2) * IBUF)

        boff = p0 + 1 + r0

        for q in range(RCH):
            offv = (boff + iota) % 8
            plsc.store_scatter(stg_v, [iota * ST_H + offv + q * NP], zeros)

        sbase = [h * ST_H + (boff + h) % 8 for h in range(H)]

        @plsc.parallel_loop(0, RCH * GPR, unroll=2)
        def group(g):
            r = g // GPR
            c = (g % GPR) * L
            ivec = idx_v[pl.ds(ibase + IMAIN + g * L, L)]
            base = ivec * HS
            dvec = r * NP + 1 + c + iota
            for h in range(H):
                vals = plsc.load_gather(tab_v, [base + h])
                plsc.store_scatter(stg_v, [dvec + sbase[h]], vals)

        rzf = (r0 > 0).astype(jnp.float32)
        for h in range(H):
            off_h = (boff + h) % 8
            ivp = plsc.load_gather(idx_v, [ibase + IPREV + N - off_h + iota])
            hvals = plsc.load_gather(tab_v, [ivp * HS + h]) * rzf
            plsc.store_scatter(
                stg_v, [h * ST_H + iota], hvals, mask=iota < off_h
            )

            @pl.when(jnp.logical_not(is_last))
            def _():
                ivn = plsc.load_gather(idx_v, [ibase + INEXT - 1 + iota])
                tvals = plsc.load_gather(tab_v, [ivn * HS + h])
                tvals = jnp.where(iota == 0, 0.0, tvals)
                plsc.store_scatter(
                    stg_v,
                    [h * ST_H + off_h + CW + iota],
                    tvals,
                    mask=iota < 8 - off_h,
                )

            @pl.when(is_last)
            def _():
                for j in range(GPR + 1):
                    plsc.store_scatter(
                        stg_v,
                        [h * ST_H + off_h + CW + j * L + iota],
                        zeros,
                        mask=(j * L + iota) < NP,
                    )

            p = p0 + h
            a = pl.multiple_of((p * NP + 1 + r0) * NP - off_h, 8)

            @pl.when(jnp.logical_not(is_last))
            def _():
                pltpu.async_copy(
                    stg_v.at[pl.ds(h * ST_H, SZ_N)],
                    out_hbm.at[pl.ds(a, SZ_N)],
                    sem_o,
                )

            @pl.when(jnp.logical_and(is_last, p < B * H - 1))
            def _():
                pltpu.sync_copy(
                    stg_v.at[pl.ds(h * ST_H, SZ_L)],
                    out_hbm.at[pl.ds(a, SZ_L)],
                )

                @pl.when(off_h == 7)
                def _():
                    pltpu.sync_copy(
                        zbuf_v.at[pl.ds(0, 8)],
                        out_hbm.at[pl.ds(a + SZ_L, 8)],
                    )

            @pl.when(jnp.logical_and(is_last, p == B * H - 1))
            def _():
                pltpu.sync_copy(
                    stg_v.at[pl.ds(h * ST_H, CW)],
                    out_hbm.at[pl.ds(a, CW)],
                )

        return carry
    lax.fori_loop(0, NCH, chunk, 0)

    @pl.when(half == 0)
    def _():
        drain_out()


@functools.partial(
    pl.kernel,
    out_type=jax.ShapeDtypeStruct((B * H * NP * NP,), jnp.float32),
    mesh=plsc.VectorSubcoreMesh(core_axis_name="c", subcore_axis_name="s"),
    compiler_params=pltpu.CompilerParams(
        use_tc_tiling_on_sc=False, needs_layout_passes=False
    ),
    scratch_types=[
        pltpu.VMEM((V * HS + L,), jnp.float32),
        pltpu.VMEM((2 * IBUF,), jnp.int32),
        pltpu.VMEM((H * ST_H,), jnp.float32),
        pltpu.VMEM((N,), jnp.float32),
        pltpu.SemaphoreType.DMA,
        pltpu.SemaphoreType.DMA,
    ],
)
def _spd(sp_hbm, tab_hbm, out_hbm, tab_v, idx_v, stg_v, zbuf_v, sem_o,
         sem_i):
    _body(sp_hbm, tab_hbm, out_hbm, tab_v, idx_v, stg_v, zbuf_v, sem_o,
          sem_i)


def kernel(spatial_pos, table):
    sp_flat = spatial_pos.astype(jnp.int32).reshape(-1)
    tab_flat = table.reshape(-1)
    out = _spd(sp_flat, tab_flat)
    return out.reshape(B, H, NP, NP)


# speedup vs baseline: 1.1915x; 1.1915x over previous
import functools

import jax
import jax.numpy as jnp
from jax import lax
from jax.experimental import pallas as pl
from jax.experimental.pallas import tpu as pltpu
from jax.experimental.pallas import tpu_sc as plsc

B, N, H, V = 16, 512, 16, 245
NP = N + 1
NC, NS, L = 2, 16, 16
NW = NC * NS
ROWS_W = (B * N) // NW
RCH = 8
NCH = ROWS_W // RCH
GPR = N // L
CW = RCH * NP
ST_H = 4632
SZ_N = CW + 8
SZ_L = CW + NP - 1
TW = V * H
HS = H + 1
IPREV, IMAIN, INEXT = 0, N, N + RCH * N
IBUF = N + RCH * N + N
SPTOT = B * N * N


def _body(sp_hbm, tab_hbm, out_hbm, tab_v, idx_v, stg_v, zbuf_v, sem_o,
          sem_i):
    cid = lax.axis_index("c")
    sid = lax.axis_index("s")
    wid = sid * NC + cid
    b = wid // 2
    half = wid % 2
    p0 = b * H

    iota = lax.iota(jnp.int32, L)
    zeros = jnp.zeros((L,), jnp.float32)
    izeros = jnp.zeros((L,), jnp.int32)

    pltpu.sync_copy(tab_hbm, stg_v.at[pl.ds(0, TW)])

    def repack(v, carry):
        row = stg_v[pl.ds(v * H, L)]
        plsc.store_scatter(tab_v, [v * HS + iota], row)
        return carry
    lax.fori_loop(0, V, repack, 0)

    for j in range(N // L):
        zbuf_v[pl.ds(j * L, L)] = zeros
        idx_v[pl.ds(IPREV + j * L, L)] = izeros
        idx_v[pl.ds(IBUF + IPREV + j * L, L)] = izeros

    @pl.when(wid == 0)
    def _():
        pltpu.sync_copy(zbuf_v, out_hbm.at[pl.ds(0, N)])

    row_base = half * ROWS_W
    idx_base = b * (N * N) + row_base * N

    def issue_idx(k, ibase):
        om = pl.multiple_of(idx_base + k * (RCH * N), 8)
        op = pl.multiple_of(jnp.maximum(om - N, 0), 8)
        on = pl.multiple_of(jnp.minimum(om + RCH * N, SPTOT - N), 8)
        pltpu.async_copy(
            sp_hbm.at[pl.ds(om, RCH * N)],
            idx_v.at[pl.ds(ibase + IMAIN, RCH * N)],
            sem_i,
        )
        pltpu.async_copy(
            sp_hbm.at[pl.ds(op, N)], idx_v.at[pl.ds(ibase + IPREV, N)], sem_i
        )
        pltpu.async_copy(
            sp_hbm.at[pl.ds(on, N)], idx_v.at[pl.ds(ibase + INEXT, N)], sem_i
        )

    def drain_idx():
        pltpu.make_async_copy(
            sp_hbm.at[pl.ds(0, RCH * N)], idx_v.at[pl.ds(IMAIN, RCH * N)],
            sem_i,
        ).wait()
        pltpu.make_async_copy(
            sp_hbm.at[pl.ds(0, N)], idx_v.at[pl.ds(IPREV, N)], sem_i
        ).wait()
        pltpu.make_async_copy(
            sp_hbm.at[pl.ds(0, N)], idx_v.at[pl.ds(INEXT, N)], sem_i
        ).wait()

    def drain_out():
        for _h in range(H):
            pltpu.make_async_copy(
                out_hbm.at[pl.ds(0, SZ_N)], stg_v.at[pl.ds(0, SZ_N)], sem_o
            ).wait()

    issue_idx(0, 0)

    def chunk(k, carry):
        r0 = row_base + k * RCH
        is_last = jnp.logical_and(half == 1, k == NCH - 1)
        ibase = (k % 2) * IBUF

        @pl.when(k > 0)
        def _():
            drain_out()

        drain_idx()

        @pl.when(k < NCH - 1)
        def _():
            issue_idx(k + 1, (1 - k % 2) * IBUF)

        boff = p0 + 1 + r0

        for q in range(RCH):
            offv = (boff + iota) % 8
            plsc.store_scatter(stg_v, [iota * ST_H + offv + q * NP], zeros)

        sbase = [h * ST_H + (boff + h) % 8 for h in range(H)]

        @plsc.parallel_loop(0, RCH * GPR, unroll=4)
        def group(g):
            r = g // GPR
            c = (g % GPR) * L
            ivec = idx_v[pl.ds(ibase + IMAIN + g * L, L)]
            base = ivec * HS
            dvec = r * NP + 1 + c + iota
            for h in range(H):
                vals = plsc.load_gather(tab_v, [base + h])
                plsc.store_scatter(stg_v, [dvec + sbase[h]], vals)

        rzf = (r0 > 0).astype(jnp.float32)
        for h in range(H):
            off_h = (boff + h) % 8
            ivp = plsc.load_gather(idx_v, [ibase + IPREV + N - off_h + iota])
            hvals = plsc.load_gather(tab_v, [ivp * HS + h]) * rzf
            plsc.store_scatter(
                stg_v, [h * ST_H + iota], hvals, mask=iota < off_h
            )

            @pl.when(jnp.logical_not(is_last))
            def _():
                ivn = plsc.load_gather(idx_v, [ibase + INEXT - 1 + iota])
                tvals = plsc.load_gather(tab_v, [ivn * HS + h])
                tvals = jnp.where(iota == 0, 0.0, tvals)
                plsc.store_scatter(
                    stg_v,
                    [h * ST_H + off_h + CW + iota],
                    tvals,
                    mask=iota < 8 - off_h,
                )

            @pl.when(is_last)
            def _():
                for j in range(GPR + 1):
                    plsc.store_scatter(
                        stg_v,
                        [h * ST_H + off_h + CW + j * L + iota],
                        zeros,
                        mask=(j * L + iota) < NP,
                    )

            p = p0 + h
            a = pl.multiple_of((p * NP + 1 + r0) * NP - off_h, 8)

            @pl.when(jnp.logical_not(is_last))
            def _():
                pltpu.async_copy(
                    stg_v.at[pl.ds(h * ST_H, SZ_N)],
                    out_hbm.at[pl.ds(a, SZ_N)],
                    sem_o,
                )

            @pl.when(jnp.logical_and(is_last, p < B * H - 1))
            def _():
                pltpu.sync_copy(
                    stg_v.at[pl.ds(h * ST_H, SZ_L)],
                    out_hbm.at[pl.ds(a, SZ_L)],
                )

                @pl.when(off_h == 7)
                def _():
                    pltpu.sync_copy(
                        zbuf_v.at[pl.ds(0, 8)],
                        out_hbm.at[pl.ds(a + SZ_L, 8)],
                    )

            @pl.when(jnp.logical_and(is_last, p == B * H - 1))
            def _():
                pltpu.sync_copy(
                    stg_v.at[pl.ds(h * ST_H, CW)],
                    out_hbm.at[pl.ds(a, CW)],
                )

        return carry
    lax.fori_loop(0, NCH, chunk, 0)

    @pl.when(half == 0)
    def _():
        drain_out()


@functools.partial(
    pl.kernel,
    out_type=jax.ShapeDtypeStruct((B * H * NP * NP,), jnp.float32),
    mesh=plsc.VectorSubcoreMesh(core_axis_name="c", subcore_axis_name="s"),
    compiler_params=pltpu.CompilerParams(
        use_tc_tiling_on_sc=False, needs_layout_passes=False
    ),
    scratch_types=[
        pltpu.VMEM((V * HS + L,), jnp.float32),
        pltpu.VMEM((2 * IBUF,), jnp.int32),
        pltpu.VMEM((H * ST_H,), jnp.float32),
        pltpu.VMEM((N,), jnp.float32),
        pltpu.SemaphoreType.DMA,
        pltpu.SemaphoreType.DMA,
    ],
)
def _spd(sp_hbm, tab_hbm, out_hbm, tab_v, idx_v, stg_v, zbuf_v, sem_o,
         sem_i):
    _body(sp_hbm, tab_hbm, out_hbm, tab_v, idx_v, stg_v, zbuf_v, sem_o,
          sem_i)


def kernel(spatial_pos, table):
    sp_flat = spatial_pos.astype(jnp.int32).reshape(-1)
    tab_flat = table.reshape(-1)
    out = _spd(sp_flat, tab_flat)
    return out.reshape(B, H, NP, NP)
